# baseline (speedup 1.0000x reference)
import functools

import jax
import jax.numpy as jnp
from jax import lax
from jax.experimental import pallas as pl
from jax.experimental.pallas import tpu as pltpu
from jax.experimental.pallas import tpu_sc as plsc

_N_USERS = 30000
_N = 50000
_NP = 50048
_H = 32
_E = 800000
_CHUNK = 128
_NSUB = 16
_CPT = -(-_E // (_CHUNK * _NSUB))
_EPAD = _CPT * _CHUNK * _NSUB
_RPT = _NP // _NSUB
_WCH = 184
_NWB = _RPT // _WCH
_LEAKY = 0.5


def _bcast_lane(vec, e):
    idx = jnp.full((16,), e, jnp.int32)
    return vec.at[idx].get(mode="promise_in_bounds")


def _pass_body(leaky, src_hbm, pk_hbm, out_hbm,
               acc, pk0, pk1, pk2, gb0, gb1, gb2, sb, obuf,
               gsem0, gsem1, gsem2, isem0, isem1, isem2):
    c = lax.axis_index("c")
    s = lax.axis_index("s")
    row0 = s * _RPT
    coff = c * _NP
    pk = (pk0, pk1, pk2)
    gb = (gb0, gb1, gb2)
    gs = (gsem0, gsem1, gsem2)
    isem = (isem0, isem1, isem2)

    zero = jnp.zeros((16,), jnp.float32)
    for i in range(_WCH):
        obuf[i, pl.ds(0, 16)] = zero
        obuf[i, pl.ds(16, 16)] = zero

    def zacc(k, carry):
        pltpu.sync_copy(obuf, acc.at[pl.ds(row0 + k * _WCH, _WCH)])
        return carry

    lax.fori_loop(0, _NWB, zacc, 0)
    plsc.subcore_barrier()

    def offset_and_gather(b):
        for g in range(_CHUNK // 16):
            pk[b][0, pl.ds(g * 16, 16)] = pk[b][0, pl.ds(g * 16, 16)] + coff
        pltpu.async_copy(src_hbm.at[pk[b].at[0]], gb[b], gs[b])

    def mul_scatter(b):
        for g in range(_CHUNK // 16):
            vv = plsc.bitcast(pk[b][2, pl.ds(g * 16, 16)], jnp.float32)
            for e in range(16):
                r = g * 16 + e
                w = _bcast_lane(vv, e)
                sb[r, pl.ds(0, 16)] = gb[b][r, pl.ds(0, 16)] * w
                sb[r, pl.ds(16, 16)] = gb[b][r, pl.ds(16, 16)] * w
        pltpu.sync_copy(sb, acc.at[pk[b].at[1]], add=True)

    chunk0 = s * _CPT
    pltpu.sync_copy(pk_hbm.at[chunk0], pk0)
    offset_and_gather(0)
    pltpu.sync_copy(pk_hbm.at[chunk0 + 1], pk1)
    offset_and_gather(1)
    pltpu.async_copy(pk_hbm.at[chunk0 + 2], pk2, isem[2])

    def step(j, carry):
        b = lax.rem(j, 3)

        def one(bb):
            pltpu.make_async_copy(src_hbm.at[pk[bb].at[0]], gb[bb],
                                  gs[bb]).wait()
            n2 = (bb + 2) % 3

            @pl.when(j + 2 < _CPT)
            def _():
                pltpu.make_async_copy(pk_hbm.at[chunk0], pk[n2],
                                      isem[n2]).wait()
                offset_and_gather(n2)
            mul_scatter(bb)

            @pl.when(j + 3 < _CPT)
            def _():
                pltpu.async_copy(pk_hbm.at[chunk0 + j + 3], pk[bb], isem[bb])

        @pl.when(b == 0)
        def _():
            one(0)

        @pl.when(b == 1)
        def _():
            one(1)

        @pl.when(b == 2)
        def _():
            one(2)

        return carry

    lax.fori_loop(0, _CPT, step, 0)
    plsc.subcore_barrier()

    def wb(k, carry):
        r0 = row0 + k * _WCH
        if leaky:
            pltpu.sync_copy(acc.at[pl.ds(r0, _WCH)], obuf)
            for i in range(_WCH):
                for h in range(2):
                    x = obuf[i, pl.ds(h * 16, 16)]
                    obuf[i, pl.ds(h * 16, 16)] = jnp.where(
                        x >= 0, x, x * _LEAKY)
            pltpu.sync_copy(obuf, out_hbm.at[pl.ds(coff + r0, _WCH)])
        else:
            pltpu.sync_copy(acc.at[pl.ds(r0, _WCH)],
                            out_hbm.at[pl.ds(coff + r0, _WCH)])
        return carry

    lax.fori_loop(0, _NWB, wb, 0)


def _conv(src, packed, leaky):
    f = pl.kernel(
        functools.partial(_pass_body, leaky),
        out_type=jax.ShapeDtypeStruct((2 * _NP, _H), jnp.float32),
        mesh=plsc.VectorSubcoreMesh(core_axis_name="c", subcore_axis_name="s",
                                    num_cores=2, num_subcores=16),
        scratch_types=(
            [pltpu.VMEM_SHARED((_NP, _H), jnp.float32)]
            + [pltpu.VMEM((3, _CHUNK), jnp.int32)] * 3
            + [pltpu.VMEM((_CHUNK, _H), jnp.float32)] * 4
            + [pltpu.VMEM((_WCH, _H), jnp.float32)]
            + [pltpu.SemaphoreType.DMA] * 6
        ),
        compiler_params=pltpu.CompilerParams(use_tc_tiling_on_sc=False, needs_layout_passes=False),
        name="hgconv_leaky" if leaky else "hgconv",
    )
    return f(src, packed)


def _pack_edges(g_idx, s_idx, vals):
    pad = _EPAD - _E
    g = jnp.pad(g_idx, (0, pad))
    s = jnp.pad(s_idx, (0, pad))
    v = lax.bitcast_convert_type(jnp.pad(vals, (0, pad)), jnp.int32)
    pk = jnp.stack([g, s, v], axis=1)
    return pk.reshape(_EPAD // _CHUNK, _CHUNK, 3).transpose(0, 2, 1)


def kernel(ego_embeddings, adj_indices, adj_values):
    rows = adj_indices[0].astype(jnp.int32)
    cols = adj_indices[1].astype(jnp.int32)
    vals = adj_values.astype(jnp.float32)
    pk_a = _pack_edges(rows, cols, vals)
    pk_b = _pack_edges(cols, rows, vals)

    x = ego_embeddings
    rpad = ((0, _NP - _N), (0, 0))
    src = jnp.concatenate(
        [jnp.pad(x[:, :_H], rpad), jnp.pad(x[:, _H:], rpad)], axis=0)

    for _ in range(2):
        t = _conv(src, pk_a, leaky=False)
        src = _conv(t, pk_b, leaky=True)

    last = jnp.concatenate([src[:_N], src[_NP:_NP + _N]], axis=1)
    return last[:_N_USERS], last[_N_USERS:]


# speedup vs baseline: 1.1383x; 1.1383x over previous
import functools

import jax
import jax.numpy as jnp
from jax import lax
from jax.experimental import pallas as pl
from jax.experimental.pallas import tpu as pltpu
from jax.experimental.pallas import tpu_sc as plsc

_N_USERS = 30000
_N = 50000
_NP = 50048
_H = 32
_E = 800000
_CHUNK = 128
_NSUB = 16
_CPT = -(-_E // (_CHUNK * _NSUB))
_EPAD = _CPT * _CHUNK * _NSUB
_RPT = _NP // _NSUB
_WCH = 184
_NWB = _RPT // _WCH
_LEAKY = 0.5


def _bcast_lane(vec, e):
    idx = jnp.full((16,), e, jnp.int32)
    return vec.at[idx].get(mode="promise_in_bounds")


def _pass_body(leaky, src_hbm, pk_hbm, out_hbm,
               acc, pk0, pk1, gb0, gb1, sb0, sb1, si0, si1, obuf,
               gsem0, gsem1, ssem0, ssem1, isem):
    c = lax.axis_index("c")
    s = lax.axis_index("s")
    row0 = s * _RPT
    coff = c * _NP
    pk = (pk0, pk1)
    gb = (gb0, gb1)
    sb = (sb0, sb1)
    si = (si0, si1)
    gs = (gsem0, gsem1)
    ss = (ssem0, ssem1)

    zero = jnp.zeros((16,), jnp.float32)
    for i in range(_WCH):
        obuf[i, pl.ds(0, 16)] = zero
        obuf[i, pl.ds(16, 16)] = zero

    def zacc(k, carry):
        pltpu.sync_copy(obuf, acc.at[pl.ds(row0 + k * _WCH, _WCH)])
        return carry

    lax.fori_loop(0, _NWB, zacc, 0)
    plsc.subcore_barrier()

    def offset_and_gather(b):
        for g in range(_CHUNK // 16):
            pk[b][0, pl.ds(g * 16, 16)] = pk[b][0, pl.ds(g * 16, 16)] + coff
        pltpu.async_copy(src_hbm.at[pk[b].at[0]], gb[b], gs[b])

    def mul_scatter(b, j):
        @pl.when(j >= 2)
        def _():
            pltpu.make_async_copy(sb[b], acc.at[si[b]], ss[b]).wait()
        for g in range(_CHUNK // 16):
            vv = plsc.bitcast(pk[b][2, pl.ds(g * 16, 16)], jnp.float32)
            si[b][pl.ds(g * 16, 16)] = pk[b][1, pl.ds(g * 16, 16)]
            for e in range(16):
                r = g * 16 + e
                w = _bcast_lane(vv, e)
                sb[b][r, pl.ds(0, 16)] = gb[b][r, pl.ds(0, 16)] * w
                sb[b][r, pl.ds(16, 16)] = gb[b][r, pl.ds(16, 16)] * w
        pltpu.async_copy(sb[b], acc.at[si[b]], ss[b], add=True)

    chunk0 = s * _CPT
    pltpu.sync_copy(pk_hbm.at[chunk0], pk0)
    offset_and_gather(0)
    pltpu.async_copy(pk_hbm.at[chunk0 + 1], pk1, isem)

    def step(j, carry):
        b = lax.rem(j, 2)

        def one(bb):
            @pl.when(j + 1 < _CPT)
            def _():
                pltpu.make_async_copy(pk_hbm.at[chunk0], pk[1 - bb],
                                      isem).wait()
                offset_and_gather(1 - bb)
            pltpu.make_async_copy(src_hbm.at[pk[bb].at[0]], gb[bb],
                                  gs[bb]).wait()
            mul_scatter(bb, j)

            @pl.when(j + 2 < _CPT)
            def _():
                pltpu.async_copy(pk_hbm.at[chunk0 + j + 2], pk[bb], isem)

        @pl.when(b == 0)
        def _():
            one(0)

        @pl.when(b == 1)
        def _():
            one(1)

        return carry

    lax.fori_loop(0, _CPT, step, 0)
    pltpu.make_async_copy(sb[0], acc.at[si[0]], ss[0]).wait()
    pltpu.make_async_copy(sb[1], acc.at[si[1]], ss[1]).wait()
    plsc.subcore_barrier()

    def wb(k, carry):
        r0 = row0 + k * _WCH
        if leaky:
            pltpu.sync_copy(acc.at[pl.ds(r0, _WCH)], obuf)
            for i in range(_WCH):
                for h in range(2):
                    x = obuf[i, pl.ds(h * 16, 16)]
                    obuf[i, pl.ds(h * 16, 16)] = jnp.where(
                        x >= 0, x, x * _LEAKY)
            pltpu.sync_copy(obuf, out_hbm.at[pl.ds(coff + r0, _WCH)])
        else:
            pltpu.sync_copy(acc.at[pl.ds(r0, _WCH)],
                            out_hbm.at[pl.ds(coff + r0, _WCH)])
        return carry

    lax.fori_loop(0, _NWB, wb, 0)


def _conv(src, packed, leaky):
    f = pl.kernel(
        functools.partial(_pass_body, leaky),
        out_type=jax.ShapeDtypeStruct((2 * _NP, _H), jnp.float32),
        mesh=plsc.VectorSubcoreMesh(core_axis_name="c", subcore_axis_name="s",
                                    num_cores=2, num_subcores=16),
        scratch_types=(
            [pltpu.VMEM_SHARED((_NP, _H), jnp.float32)]
            + [pltpu.VMEM((3, _CHUNK), jnp.int32)] * 2
            + [pltpu.VMEM((_CHUNK, _H), jnp.float32)] * 4
            + [pltpu.VMEM((_CHUNK,), jnp.int32)] * 2
            + [pltpu.VMEM((_WCH, _H), jnp.float32)]
            + [pltpu.SemaphoreType.DMA] * 5
        ),
        compiler_params=pltpu.CompilerParams(use_tc_tiling_on_sc=False, needs_layout_passes=False),
        name="hgconv_leaky" if leaky else "hgconv",
    )
    return f(src, packed)


def _pack_edges(g_idx, s_idx, vals):
    pad = _EPAD - _E
    g = jnp.pad(g_idx, (0, pad))
    s = jnp.pad(s_idx, (0, pad))
    v = lax.bitcast_convert_type(jnp.pad(vals, (0, pad)), jnp.int32)
    pk = jnp.stack([g, s, v], axis=1)
    return pk.reshape(_EPAD // _CHUNK, _CHUNK, 3).transpose(0, 2, 1)


def kernel(ego_embeddings, adj_indices, adj_values):
    rows = adj_indices[0].astype(jnp.int32)
    cols = adj_indices[1].astype(jnp.int32)
    vals = adj_values.astype(jnp.float32)
    pk_a = _pack_edges(rows, cols, vals)
    pk_b = _pack_edges(cols, rows, vals)

    x = ego_embeddings
    rpad = ((0, _NP - _N), (0, 0))
    src = jnp.concatenate(
        [jnp.pad(x[:, :_H], rpad), jnp.pad(x[:, _H:], rpad)], axis=0)

    for _ in range(2):
        t = _conv(src, pk_a, leaky=False)
        src = _conv(t, pk_b, leaky=True)

    last = jnp.concatenate([src[:_N], src[_NP:_NP + _N]], axis=1)
    return last[:_N_USERS], last[_N_USERS:]
